# baseline (speedup 1.0000x reference)
import functools

import jax
import jax.numpy as jnp
from jax.experimental import pallas as pl
from jax.experimental.pallas import tpu as pltpu


def _row_sums_accumulate(src_ref, dacc, w, d):
    sums = {}
    for c in range(0, w, d):
        e = min(c + d, w)
        p = src_ref[:, c:e].astype(jnp.float32)
        width = e - c
        sums[width] = p if width not in sums else sums[width] + p
    for width, val in sums.items():
        dacc[:, :width] += val


def _layer0_body(x_ref, a_ref, w0_ref, b0_ref, w1_ref, b1_ref,
                 abf_ref, xlin_ref, xbf_ref, dacc,
                 m_acc, *, k_blocks, valid_last):
    k = pl.program_id(0)
    d = x_ref.shape[1]

    @pl.when(k == 0)
    def _():
        dacc[...] = jnp.zeros(dacc.shape, dacc.dtype)
        xlin_ref[...] = jax.lax.dot_general(
            x_ref[...].astype(jnp.bfloat16), w0_ref[...].astype(jnp.bfloat16),
            (((1,), (1,)), ((), ())),
            preferred_element_type=jnp.float32,
        ) + b0_ref[...]
        xbf_ref[...] = xlin_ref[...].astype(jnp.bfloat16)

    def step(w):
        abf_ref[:, :w] = a_ref[:, :w].astype(jnp.bfloat16)
        m01 = jax.lax.dot_general(
            abf_ref[:, :w], xbf_ref[...], (((0,), (0,)), ((), ())),
            preferred_element_type=jnp.float32)
        contrib = jax.lax.dot_general(
            abf_ref[:, :w], m01.astype(jnp.bfloat16), (((1,), (0,)), ((), ())),
            preferred_element_type=jnp.float32)

        @pl.when(k == 0)
        def _():
            m_acc[...] = contrib

        @pl.when(k > 0)
        def _():
            m_acc[...] += contrib

        _row_sums_accumulate(abf_ref, dacc, w, d)

    full_bk = a_ref.shape[1]
    if valid_last == full_bk:
        step(full_bk)
    else:
        @pl.when(k < k_blocks - 1)
        def _():
            step(full_bk)

        @pl.when(k == k_blocks - 1)
        def _():
            step(valid_last)

    @pl.when(k == k_blocks - 1)
    def _():
        dv = jnp.sum(dacc[...], axis=1, keepdims=True)
        x2 = xlin_ref[...] + m_acc[...] / dv
        dacc[:, 0:1] = dv
        xlin_ref[...] = jax.lax.dot_general(
            x2.astype(jnp.bfloat16), w1_ref[...].astype(jnp.bfloat16),
            (((1,), (1,)), ((), ())),
            preferred_element_type=jnp.float32,
        ) + b1_ref[...]
        xbf_ref[...] = xlin_ref[...].astype(jnp.bfloat16)


def _layer1_body(a_ref, xlin_ref, xbf_ref, deg_ref, wout_ref, bout_ref,
                 out_ref, m_acc, *, k_blocks, valid_last):
    k = pl.program_id(0)

    def step(w):
        m01 = jax.lax.dot_general(
            a_ref[:, :w], xbf_ref[...], (((0,), (0,)), ((), ())),
            preferred_element_type=jnp.float32)
        contrib = jax.lax.dot_general(
            a_ref[:, :w], m01.astype(jnp.bfloat16), (((1,), (0,)), ((), ())),
            preferred_element_type=jnp.float32)

        @pl.when(k == 0)
        def _():
            m_acc[...] = contrib

        @pl.when(k > 0)
        def _():
            m_acc[...] += contrib

    full_bk = a_ref.shape[1]
    if valid_last == full_bk:
        step(full_bk)
    else:
        @pl.when(k < k_blocks - 1)
        def _():
            step(full_bk)

        @pl.when(k == k_blocks - 1)
        def _():
            step(valid_last)

    @pl.when(k == k_blocks - 1)
    def _():
        x2 = xlin_ref[...] + m_acc[...] / deg_ref[:, 0:1]
        pooled = jnp.max(x2, axis=0, keepdims=True)
        logit = jnp.sum(pooled * wout_ref[...], axis=1, keepdims=True)
        out_ref[...] = jax.nn.sigmoid(logit + bout_ref[...])


@jax.jit
def kernel(x_1, incidence_1, W0, b0, W1, b1, W_out, b_out):
    n, d = x_1.shape

    bk0 = min(256, n)
    k0 = -(-n // bk0)
    valid0 = n - (k0 - 1) * bk0

    abf, xlin, xbf, deg = pl.pallas_call(
        functools.partial(_layer0_body, k_blocks=k0, valid_last=valid0),
        grid=(k0,),
        in_specs=[
            pl.BlockSpec((n, d), lambda k: (0, 0)),
            pl.BlockSpec((n, bk0), lambda k: (0, k)),
            pl.BlockSpec((d, d), lambda k: (0, 0)),
            pl.BlockSpec((1, d), lambda k: (0, 0)),
            pl.BlockSpec((d, d), lambda k: (0, 0)),
            pl.BlockSpec((1, d), lambda k: (0, 0)),
        ],
        out_specs=[
            pl.BlockSpec((n, bk0), lambda k: (0, k)),
            pl.BlockSpec((n, d), lambda k: (0, 0)),
            pl.BlockSpec((n, d), lambda k: (0, 0)),
            pl.BlockSpec((n, d), lambda k: (0, 0)),
        ],
        out_shape=[
            jax.ShapeDtypeStruct((n, n), jnp.bfloat16),
            jax.ShapeDtypeStruct((n, d), jnp.float32),
            jax.ShapeDtypeStruct((n, d), jnp.bfloat16),
            jax.ShapeDtypeStruct((n, d), jnp.float32),
        ],
        scratch_shapes=[
            pltpu.VMEM((n, d), jnp.float32),
        ],
        compiler_params=pltpu.CompilerParams(
            dimension_semantics=("arbitrary",),
            vmem_limit_bytes=60 * 1024 * 1024,
        ),
    )(x_1, incidence_1, W0, b0.reshape(1, d), W1, b1.reshape(1, d))

    bk1 = min(768, n)
    k1 = -(-n // bk1)
    valid1 = n - (k1 - 1) * bk1

    out = pl.pallas_call(
        functools.partial(_layer1_body, k_blocks=k1, valid_last=valid1),
        grid=(k1,),
        in_specs=[
            pl.BlockSpec((n, bk1), lambda k: (0, k)),
            pl.BlockSpec((n, d), lambda k: (0, 0)),
            pl.BlockSpec((n, d), lambda k: (0, 0)),
            pl.BlockSpec((n, d), lambda k: (0, 0)),
            pl.BlockSpec((1, d), lambda k: (0, 0)),
            pl.BlockSpec((1, 1), lambda k: (0, 0)),
        ],
        out_specs=pl.BlockSpec((1, 1), lambda k: (0, 0)),
        out_shape=jax.ShapeDtypeStruct((1, 1), jnp.float32),
        scratch_shapes=[
            pltpu.VMEM((n, d), jnp.float32),
        ],
        compiler_params=pltpu.CompilerParams(
            dimension_semantics=("arbitrary",),
            vmem_limit_bytes=60 * 1024 * 1024,
        ),
    )(abf, xlin, xbf, deg, W_out, b_out.reshape(1, 1))
    return out.reshape(1)


# speedup vs baseline: 1.1765x; 1.1765x over previous
import functools

import jax
import jax.numpy as jnp
from jax.experimental import pallas as pl
from jax.experimental.pallas import tpu as pltpu


def _unisage_body(x_ref, a_ref, w_ref, b_ref, wout_ref, bout_ref,
                  out_ref, x_state, x_bf, a_bf, m_acc, dacc,
                  *, n_layers, k_blocks, valid_last):
    l = pl.program_id(0)
    k = pl.program_id(1)
    d = x_ref.shape[1]

    @pl.when((l == 0) & (k == 0))
    def _():
        x_state[...] = x_ref[...]
        dacc[...] = jnp.zeros(dacc.shape, dacc.dtype)

    @pl.when(k == 0)
    def _():
        x_state[...] = jax.lax.dot_general(
            x_state[...].astype(jnp.bfloat16), w_ref[0].astype(jnp.bfloat16),
            (((1,), (1,)), ((), ())),
            preferred_element_type=jnp.float32,
        ) + b_ref[0]
        x_bf[...] = x_state[...].astype(jnp.bfloat16)

    def cast_step(w):
        buf = k % 2
        a_bf[buf, :, :w] = a_ref[:, :w].astype(jnp.bfloat16)

        @pl.when(l == 0)
        def _():
            sums = {}
            for c in range(0, w, d):
                e = min(c + d, w)
                p = a_bf[buf, :, c:e].astype(jnp.float32)
                width = e - c
                sums[width] = p if width not in sums else sums[width] + p
            for width, val in sums.items():
                dacc[:, :width] += val

    def mm_step(w):
        buf = (k - 1) % 2
        m01 = jax.lax.dot_general(
            a_bf[buf, :, :w], x_bf[...], (((0,), (0,)), ((), ())),
            preferred_element_type=jnp.float32)
        contrib = jax.lax.dot_general(
            a_bf[buf, :, :w], m01.astype(jnp.bfloat16), (((1,), (0,)), ((), ())),
            preferred_element_type=jnp.float32)

        @pl.when(k == 1)
        def _():
            m_acc[...] = contrib

        @pl.when(k > 1)
        def _():
            m_acc[...] += contrib

    full_bk = a_ref.shape[1]
    if valid_last == full_bk:
        @pl.when(k < k_blocks)
        def _():
            cast_step(full_bk)

        @pl.when(k > 0)
        def _():
            mm_step(full_bk)
    else:
        @pl.when(k < k_blocks - 1)
        def _():
            cast_step(full_bk)

        @pl.when(k == k_blocks - 1)
        def _():
            cast_step(valid_last)

        @pl.when((k > 0) & (k < k_blocks))
        def _():
            mm_step(full_bk)

        @pl.when(k == k_blocks)
        def _():
            mm_step(valid_last)

    @pl.when(k == k_blocks)
    def _():
        @pl.when(l == 0)
        def _():
            dv = jnp.sum(dacc[...], axis=1, keepdims=True)
            x_state[...] = x_state[...] + m_acc[...] / dv
            dacc[:, 0:1] = dv

        @pl.when(l > 0)
        def _():
            x_state[...] = x_state[...] + m_acc[...] / dacc[:, 0:1]

    @pl.when((l == n_layers - 1) & (k == k_blocks))
    def _():
        pooled = jnp.max(x_state[...], axis=0, keepdims=True)
        logit = jnp.sum(pooled * wout_ref[...], axis=1, keepdims=True)
        out_ref[...] = jax.nn.sigmoid(logit + bout_ref[...])


@jax.jit
def kernel(x_1, incidence_1, W0, b0, W1, b1, W_out, b_out):
    n, d = x_1.shape
    n_layers = 2
    bk = min(256, n)
    k_blocks = -(-n // bk)
    valid_last = n - (k_blocks - 1) * bk

    ws = jnp.stack([W0, W1])
    bs = jnp.stack([b0, b1]).reshape(n_layers, 1, d)
    bout = b_out.reshape(1, 1)

    kb = k_blocks
    grid = (n_layers, k_blocks + 1)
    out = pl.pallas_call(
        functools.partial(_unisage_body, n_layers=n_layers,
                          k_blocks=k_blocks, valid_last=valid_last),
        grid=grid,
        in_specs=[
            pl.BlockSpec((n, d), lambda l, k: (0, 0)),
            pl.BlockSpec((n, bk), lambda l, k: (0, jnp.minimum(k, kb - 1))),
            pl.BlockSpec((1, d, d), lambda l, k: (l, 0, 0)),
            pl.BlockSpec((1, 1, d), lambda l, k: (l, 0, 0)),
            pl.BlockSpec((1, d), lambda l, k: (0, 0)),
            pl.BlockSpec((1, 1), lambda l, k: (0, 0)),
        ],
        out_specs=pl.BlockSpec((1, 1), lambda l, k: (0, 0)),
        out_shape=jax.ShapeDtypeStruct((1, 1), jnp.float32),
        scratch_shapes=[
            pltpu.VMEM((n, d), jnp.float32),
            pltpu.VMEM((n, d), jnp.bfloat16),
            pltpu.VMEM((2, n, bk), jnp.bfloat16),
            pltpu.VMEM((n, d), jnp.float32),
            pltpu.VMEM((n, d), jnp.float32),
        ],
        compiler_params=pltpu.CompilerParams(
            dimension_semantics=("arbitrary", "arbitrary"),
            vmem_limit_bytes=60 * 1024 * 1024,
        ),
    )(x_1, incidence_1, ws, bs, W_out, bout)
    return out.reshape(1)
